# baseline (speedup 1.0000x reference)
import functools

import jax
import jax.numpy as jnp
from jax import lax
from jax.experimental import pallas as pl
from jax.experimental.pallas import tpu as pltpu
from jax.experimental.pallas import tpu_sc as plsc

L = 16
NB = 3
QP = 201


def _cumsum16(v):
    iota = lax.iota(jnp.int32, L)
    for k in (1, 2, 4, 8):
        idx = jnp.maximum(iota - k, 0)
        v = v + jnp.where(iota >= k, v[idx], 0)
    return v


def _sc_info():
    try:
        info = plsc.get_sparse_core_info()
        return info.num_cores, info.num_subcores
    except Exception:
        return 2, 16


@functools.lru_cache(maxsize=None)
def _make_sc_call(B, S, V, D):
    NC, NS = _sc_info()
    NW = NC * NS
    assert B % NW == 0 and S == 200 and D == 64
    RPW = B // NW
    NCH = 13
    NQ = S // 4
    NQF = 56
    QD = 4 * D

    mesh = plsc.VectorSubcoreMesh(core_axis_name="c", subcore_axis_name="s")

    @functools.partial(
        pl.kernel,
        out_type=jax.ShapeDtypeStruct((B, S, D), jnp.float32),
        mesh=mesh,
        scratch_types=[
            pltpu.VMEM((RPW * 2, 128), jnp.int32),
            pltpu.VMEM((NB, 64), jnp.int32),
            pltpu.VMEM((NB * S, D), jnp.float32),
            pltpu.VMEM((NB * NQF, QD), jnp.float32),
            pltpu.SemaphoreType.DMA((NB,)),
            pltpu.SemaphoreType.DMA((NB,)),
            pltpu.SemaphoreType.DMA((NB,)),
        ],
        compiler_params=pltpu.CompilerParams(use_tc_tiling_on_sc=False),
    )
    def sc_embed(xp_hbm, tok_hbm, qtab_hbm, out_hbm,
                 idx_all, posidx, rows, posq,
                 sem_pos, sem_tok, sem_out):
        cid = lax.axis_index("c")
        sid = lax.axis_index("s")
        wid = sid * NC + cid

        pltpu.sync_copy(xp_hbm.at[pl.ds(wid * (RPW * 2), RPW * 2)], idx_all)

        iota = lax.iota(jnp.int32, L)
        g_m = jnp.minimum(iota * 4 + 3, L - 1)
        g_c = jnp.minimum(iota * 4, L - 1)

        def tok_descs(b, row):
            return (
                pltpu.make_async_copy(tok_hbm.at[idx_all.at[2 * row]],
                                      rows.at[pl.ds(b * S, 128)],
                                      sem_tok.at[b]),
                pltpu.make_async_copy(
                    tok_hbm.at[idx_all.at[2 * row + 1, pl.ds(0, S - 128)]],
                    rows.at[pl.ds(b * S + 128, S - 128)], sem_tok.at[b]),
            )

        def pos_desc(b, row):
            return pltpu.make_async_copy(
                qtab_hbm.at[posidx.at[b, pl.ds(0, NQF)]],
                posq.at[pl.ds(b * NQF, NQF)], sem_pos.at[b])

        def out_desc(b, row):
            return pltpu.make_async_copy(rows.at[pl.ds(b * S, S)],
                                         out_hbm.at[wid * RPW + row],
                                         sem_out.at[b])

        def stage_a(t, b):
            @pl.when(jnp.logical_and(t < RPW, t >= NB))
            def _():
                out_desc(b, t - NB).wait()

            @pl.when(t < RPW)
            def _():
                for d in tok_descs(b, t):
                    d.start()
                carry = jnp.int32(0)
                qis = []
                for k in range(NCH):
                    part, c = (0, k) if k < 8 else (1, k - 8)
                    tok = idx_all[2 * t + part, pl.ds(c * L, L)]
                    nz = jnp.where(tok != 0, 1, 0).astype(jnp.int32)
                    cs = _cumsum16(nz) + carry
                    carry = cs[L - 1]
                    excl = cs - nz
                    w = nz << (iota & 3)
                    u = w + jnp.where(iota % 4 >= 1,
                                      w[jnp.maximum(iota - 1, 0)], 0)
                    u = u + jnp.where(iota % 4 >= 2,
                                      u[jnp.maximum(iota - 2, 0)], 0)
                    qis.append(u[g_m] * QP + excl[g_c])
                for gi in range(4):
                    ch = qis[4 * gi:4 * gi + 4]
                    res = jnp.where(iota < 4, ch[0], 0)
                    for j in range(1, len(ch)):
                        sh = ch[j][jnp.maximum(iota - 4 * j, 0)]
                        res = res + jnp.where(
                            jnp.logical_and(iota >= 4 * j, iota < 4 * j + 4),
                            sh, 0)
                    posidx[b, pl.ds(16 * gi, L)] = res
                pos_desc(b, t).start()

        def stage_b(j, b):
            @pl.when(jnp.logical_and(j >= 0, j < RPW))
            def _():
                for d in tok_descs(b, j):
                    d.wait()
                pos_desc(b, j).wait()

                def add_quad(q, dummy):
                    for v in range(16):
                        r = b * S + 4 * q + v // 4
                        cc = (v % 4) * L
                        rows[r, pl.ds(cc, L)] = (
                            rows[r, pl.ds(cc, L)]
                            + posq[b * NQF + q, pl.ds(v * L, L)])
                    return dummy

                lax.fori_loop(0, NQ, add_quad, jnp.int32(0))
                out_desc(b, j).start()

        def step(g, dummy):
            for b in range(NB):
                t = g * NB + b
                stage_a(t, b)
                stage_b(t - 2, (b - 2) % NB)
            return dummy

        lax.fori_loop(0, (RPW + 2 + NB - 1) // NB, step, jnp.int32(0))
        for i in range(NB):
            j = RPW - NB + i
            out_desc(j % NB, j).wait()

    return sc_embed


def _quad_table(pos_table):
    m = jnp.arange(16)
    jj = jnp.arange(4)
    bits = (m[:, None] >> jj[None, :]) & 1
    rank = jnp.cumsum(bits, axis=1)
    c = jnp.arange(QP)
    pidx = jnp.minimum(c[None, :, None] + rank[:, None, :], QP - 1)
    quad = jnp.where(bits[:, None, :, None] == 1,
                     pos_table[pidx], 0.0)
    return quad.reshape(16 * QP, 4 * pos_table.shape[1])


def kernel(x, tok_table, pos_table):
    B, S = x.shape
    V, D = tok_table.shape
    xp = jnp.pad(x, ((0, 0), (0, 256 - S))).reshape(B * 2, 128)
    qtab = _quad_table(pos_table)
    out = _make_sc_call(B, S, V, D)(xp, tok_table, qtab)
    return out, (x == 0)


# speedup vs baseline: 1.0020x; 1.0020x over previous
import functools

import jax
import jax.numpy as jnp
from jax import lax
from jax.experimental import pallas as pl
from jax.experimental.pallas import tpu as pltpu
from jax.experimental.pallas import tpu_sc as plsc

L = 16
NB = 3
QP = 201


def _cumsum16(v):
    iota = lax.iota(jnp.int32, L)
    for k in (1, 2, 4, 8):
        idx = jnp.maximum(iota - k, 0)
        v = v + jnp.where(iota >= k, v[idx], 0)
    return v


def _sc_info():
    try:
        info = plsc.get_sparse_core_info()
        return info.num_cores, info.num_subcores
    except Exception:
        return 2, 16


@functools.lru_cache(maxsize=None)
def _make_sc_call(B, S, V, D):
    NC, NS = _sc_info()
    NW = NC * NS
    assert B % NW == 0 and S == 200 and D == 64
    RPW = B // NW
    NCH = 13
    NQ = S // 4
    NQF = 56
    QD = 4 * D

    mesh = plsc.VectorSubcoreMesh(core_axis_name="c", subcore_axis_name="s")

    @functools.partial(
        pl.kernel,
        out_type=jax.ShapeDtypeStruct((B, S, D), jnp.float32),
        mesh=mesh,
        scratch_types=[
            pltpu.VMEM((RPW * 2, 128), jnp.int32),
            pltpu.VMEM((NB, 64), jnp.int32),
            pltpu.VMEM((NB * S, D), jnp.float32),
            pltpu.VMEM((NB * NQF, QD), jnp.float32),
            pltpu.SemaphoreType.DMA((NB,)),
            pltpu.SemaphoreType.DMA((NB,)),
            pltpu.SemaphoreType.DMA((NB,)),
        ],
        compiler_params=pltpu.CompilerParams(use_tc_tiling_on_sc=False),
    )
    def sc_embed(xp_hbm, tok_hbm, qtab_hbm, out_hbm,
                 idx_all, posidx, rows, posq,
                 sem_pos, sem_tok, sem_out):
        cid = lax.axis_index("c")
        sid = lax.axis_index("s")
        wid = sid * NC + cid

        pltpu.sync_copy(xp_hbm.at[pl.ds(wid * (RPW * 2), RPW * 2)], idx_all)

        iota = lax.iota(jnp.int32, L)
        g_m = jnp.minimum(iota * 4 + 3, L - 1)
        g_c = jnp.minimum(iota * 4, L - 1)

        def tok_descs(b, row):
            return (
                pltpu.make_async_copy(tok_hbm.at[idx_all.at[2 * row]],
                                      rows.at[pl.ds(b * S, 128)],
                                      sem_tok.at[b]),
                pltpu.make_async_copy(
                    tok_hbm.at[idx_all.at[2 * row + 1, pl.ds(0, S - 128)]],
                    rows.at[pl.ds(b * S + 128, S - 128)], sem_tok.at[b]),
            )

        def pos_desc(b, row):
            return pltpu.make_async_copy(
                qtab_hbm.at[posidx.at[b, pl.ds(0, NQF)]],
                posq.at[pl.ds(b * NQF, NQF)], sem_pos.at[b])

        def out_desc(b, row):
            return pltpu.make_async_copy(rows.at[pl.ds(b * S, S)],
                                         out_hbm.at[wid * RPW + row],
                                         sem_out.at[b])

        def stage_a(t, b):
            @pl.when(jnp.logical_and(t < RPW, t >= NB))
            def _():
                out_desc(b, t - NB).wait()

            @pl.when(t < RPW)
            def _():
                for d in tok_descs(b, t):
                    d.start()
                carry = jnp.int32(0)
                qis = []
                for k in range(NCH):
                    part, c = (0, k) if k < 8 else (1, k - 8)
                    tok = idx_all[2 * t + part, pl.ds(c * L, L)]
                    nz = jnp.where(tok != 0, 1, 0).astype(jnp.int32)
                    cs = _cumsum16(nz) + carry
                    carry = cs[L - 1]
                    excl = cs - nz
                    w = nz << (iota & 3)
                    u = w + jnp.where(iota % 4 >= 1,
                                      w[jnp.maximum(iota - 1, 0)], 0)
                    u = u + jnp.where(iota % 4 >= 2,
                                      u[jnp.maximum(iota - 2, 0)], 0)
                    qis.append(u[g_m] * QP + excl[g_c])
                for gi in range(4):
                    ch = qis[4 * gi:4 * gi + 4]
                    res = jnp.where(iota < 4, ch[0], 0)
                    for j in range(1, len(ch)):
                        sh = ch[j][jnp.maximum(iota - 4 * j, 0)]
                        res = res + jnp.where(
                            jnp.logical_and(iota >= 4 * j, iota < 4 * j + 4),
                            sh, 0)
                    posidx[b, pl.ds(16 * gi, L)] = res
                pos_desc(b, t).start()

        def stage_b(j, b):
            @pl.when(jnp.logical_and(j >= 0, j < RPW))
            def _():
                for d in tok_descs(b, j):
                    d.wait()
                pos_desc(b, j).wait()

                def add_quad(q, dummy):
                    for v in range(16):
                        r = b * S + 4 * q + v // 4
                        cc = (v % 4) * L
                        plsc.addupdate(rows.at[r, pl.ds(cc, L)],
                                       posq[b * NQF + q, pl.ds(v * L, L)])
                    return dummy

                lax.fori_loop(0, NQ, add_quad, jnp.int32(0))
                out_desc(b, j).start()

        def step(g, dummy):
            for b in range(NB):
                t = g * NB + b
                stage_a(t, b)
                stage_b(t - 2, (b - 2) % NB)
            return dummy

        lax.fori_loop(0, (RPW + 2 + NB - 1) // NB, step, jnp.int32(0))
        for i in range(NB):
            j = RPW - NB + i
            out_desc(j % NB, j).wait()

    return sc_embed


def _quad_table(pos_table):
    m = jnp.arange(16)
    jj = jnp.arange(4)
    bits = (m[:, None] >> jj[None, :]) & 1
    rank = jnp.cumsum(bits, axis=1)
    c = jnp.arange(QP)
    pidx = jnp.minimum(c[None, :, None] + rank[:, None, :], QP - 1)
    quad = jnp.where(bits[:, None, :, None] == 1,
                     pos_table[pidx], 0.0)
    return quad.reshape(16 * QP, 4 * pos_table.shape[1])


def kernel(x, tok_table, pos_table):
    B, S = x.shape
    V, D = tok_table.shape
    xp = jnp.pad(x, ((0, 0), (0, 256 - S))).reshape(B * 2, 128)
    qtab = _quad_table(pos_table)
    out = _make_sc_call(B, S, V, D)(xp, tok_table, qtab)
    return out, (x == 0)


# speedup vs baseline: 1.9133x; 1.9096x over previous
import functools

import jax
import jax.numpy as jnp
from jax import lax
from jax.experimental import pallas as pl
from jax.experimental.pallas import tpu as pltpu
from jax.experimental.pallas import tpu_sc as plsc

L = 16
NB = 6


def _cumsum16(v):
    iota = lax.iota(jnp.int32, L)
    for k in (1, 2, 4, 8):
        idx = jnp.maximum(iota - k, 0)
        v = v + jnp.where(iota >= k, v[idx], 0)
    return v


def _sc_info():
    try:
        info = plsc.get_sparse_core_info()
        return info.num_cores, info.num_subcores
    except Exception:
        return 2, 16


@functools.lru_cache(maxsize=None)
def _make_sc_call(B, S, V, D, P):
    NC, NS = _sc_info()
    NW = NC * NS
    assert B % NW == 0
    RPW = B // NW
    S0 = 128
    S1 = S - S0
    NCH0 = S0 // L
    NCH1 = (S1 + L - 1) // L

    mesh = plsc.VectorSubcoreMesh(core_axis_name="c", subcore_axis_name="s")

    @functools.partial(
        pl.kernel,
        out_type=jax.ShapeDtypeStruct((B, S, D), jnp.float32),
        mesh=mesh,
        scratch_types=[
            pltpu.VMEM((RPW * 2, S0), jnp.int32),
            pltpu.VMEM((NB * 2, S0), jnp.int32),
            pltpu.VMEM((NB * S, D), jnp.float32),
            pltpu.VMEM_SHARED((P, D), jnp.float32),
            pltpu.SemaphoreType.DMA((NB,)),
            pltpu.SemaphoreType.DMA((NB,)),
            pltpu.SemaphoreType.DMA((NB,)),
        ],
        compiler_params=pltpu.CompilerParams(use_tc_tiling_on_sc=False),
    )
    def sc_embed(xp_hbm, tok_hbm, pos_hbm, out_hbm,
                 idx_all, posidx, rows, pos_sh, sem_pos, sem_tok, sem_out):
        cid = lax.axis_index("c")
        sid = lax.axis_index("s")
        wid = sid * NC + cid

        @pl.when(sid == 0)
        def _():
            pltpu.sync_copy(pos_hbm, pos_sh)

        pltpu.sync_copy(xp_hbm.at[pl.ds(wid * (RPW * 2), RPW * 2)], idx_all)
        plsc.subcore_barrier()

        def pos_descs(b, row):
            return (
                pltpu.make_async_copy(pos_sh.at[posidx.at[2 * b]],
                                      rows.at[pl.ds(b * S, S0)],
                                      sem_pos.at[b]),
                pltpu.make_async_copy(
                    pos_sh.at[posidx.at[2 * b + 1, pl.ds(0, S1)]],
                    rows.at[pl.ds(b * S + S0, S1)], sem_pos.at[b]),
            )

        def tok_descs(b, row):
            return (
                pltpu.make_async_copy(tok_hbm.at[idx_all.at[2 * row]],
                                      rows.at[pl.ds(b * S, S0)],
                                      sem_tok.at[b]),
                pltpu.make_async_copy(
                    tok_hbm.at[idx_all.at[2 * row + 1, pl.ds(0, S1)]],
                    rows.at[pl.ds(b * S + S0, S1)], sem_tok.at[b]),
            )

        def out_desc(b, row):
            return pltpu.make_async_copy(rows.at[pl.ds(b * S, S)],
                                         out_hbm.at[wid * RPW + row],
                                         sem_out.at[b])

        def stage_a(t, b):
            @pl.when(jnp.logical_and(t < RPW, t >= NB))
            def _():
                out_desc(b, t - NB).wait()

            @pl.when(t < RPW)
            def _():
                for d in tok_descs(b, t):
                    d.start()
                carry = jnp.int32(0)
                for part, nch in ((0, NCH0), (1, NCH1)):
                    r = 2 * t + part
                    for c in range(nch):
                        tok = idx_all[r, pl.ds(c * L, L)]
                        nz = jnp.where(tok != 0, 1, 0).astype(jnp.int32)
                        cs = _cumsum16(nz) + carry
                        posidx[2 * b + part, pl.ds(c * L, L)] = jnp.where(
                            tok == 0, 0, cs)
                        carry = cs[L - 1]

        def stage_b(j, b):
            @pl.when(jnp.logical_and(j >= 0, j < RPW))
            def _():
                for d in tok_descs(b, j):
                    d.wait()
                for d in pos_descs(b, j):
                    d.start(add=True)

        def stage_c(j, b):
            @pl.when(jnp.logical_and(j >= 0, j < RPW))
            def _():
                for d in pos_descs(b, j):
                    d.wait()
                out_desc(b, j).start()

        def step(g, dummy):
            for b in range(NB):
                t = g * NB + b
                stage_a(t, b)
                stage_b(t - 1, (b - 1) % NB)
                stage_c(t - 2, (b - 2) % NB)
            return dummy

        lax.fori_loop(0, (RPW + 2 + NB - 1) // NB, step, jnp.int32(0))
        for i in range(NB):
            j = RPW - NB + i
            out_desc(j % NB, j).wait()

    return sc_embed


def kernel(x, tok_table, pos_table):
    B, S = x.shape
    V, D = tok_table.shape
    P = pos_table.shape[0]
    xp = jnp.pad(x, ((0, 0), (0, 256 - S))).reshape(B * 2, 128)
    out = _make_sc_call(B, S, V, D, P)(xp, tok_table, pos_table)
    return out, (x == 0)


# speedup vs baseline: 1.9147x; 1.0007x over previous
import functools

import jax
import jax.numpy as jnp
from jax import lax
from jax.experimental import pallas as pl
from jax.experimental.pallas import tpu as pltpu
from jax.experimental.pallas import tpu_sc as plsc

L = 16
NB = 4


def _cumsum16(v):
    iota = lax.iota(jnp.int32, L)
    for k in (1, 2, 4, 8):
        idx = jnp.maximum(iota - k, 0)
        v = v + jnp.where(iota >= k, v[idx], 0)
    return v


def _sc_info():
    try:
        info = plsc.get_sparse_core_info()
        return info.num_cores, info.num_subcores
    except Exception:
        return 2, 16


@functools.lru_cache(maxsize=None)
def _make_sc_call(B, S, V, D, P):
    NC, NS = _sc_info()
    NW = NC * NS
    assert B % NW == 0
    RPW = B // NW
    S0 = 128
    S1 = S - S0
    NCH0 = S0 // L
    NCH1 = (S1 + L - 1) // L

    mesh = plsc.VectorSubcoreMesh(core_axis_name="c", subcore_axis_name="s")

    @functools.partial(
        pl.kernel,
        out_type=jax.ShapeDtypeStruct((B, S, D), jnp.float32),
        mesh=mesh,
        scratch_types=[
            pltpu.VMEM((RPW * 2, S0), jnp.int32),
            pltpu.VMEM((NB * 2, S0), jnp.int32),
            pltpu.VMEM((NB * S, D), jnp.float32),
            pltpu.VMEM_SHARED((P, D), jnp.float32),
            pltpu.SemaphoreType.DMA((NB,)),
            pltpu.SemaphoreType.DMA((NB,)),
            pltpu.SemaphoreType.DMA((NB,)),
        ],
        compiler_params=pltpu.CompilerParams(use_tc_tiling_on_sc=False),
    )
    def sc_embed(xp_hbm, tok_hbm, pos_hbm, out_hbm,
                 idx_all, posidx, rows, pos_sh, sem_pos, sem_tok, sem_out):
        cid = lax.axis_index("c")
        sid = lax.axis_index("s")
        wid = sid * NC + cid

        @pl.when(sid == 0)
        def _():
            pltpu.sync_copy(pos_hbm, pos_sh)

        pltpu.sync_copy(xp_hbm.at[pl.ds(wid * (RPW * 2), RPW * 2)], idx_all)
        plsc.subcore_barrier()

        def pos_descs(b, row):
            return (
                pltpu.make_async_copy(pos_sh.at[posidx.at[2 * b]],
                                      rows.at[pl.ds(b * S, S0)],
                                      sem_pos.at[b]),
                pltpu.make_async_copy(
                    pos_sh.at[posidx.at[2 * b + 1, pl.ds(0, S1)]],
                    rows.at[pl.ds(b * S + S0, S1)], sem_pos.at[b]),
            )

        def tok_descs(b, row):
            return (
                pltpu.make_async_copy(tok_hbm.at[idx_all.at[2 * row]],
                                      rows.at[pl.ds(b * S, S0)],
                                      sem_tok.at[b]),
                pltpu.make_async_copy(
                    tok_hbm.at[idx_all.at[2 * row + 1, pl.ds(0, S1)]],
                    rows.at[pl.ds(b * S + S0, S1)], sem_tok.at[b]),
            )

        def out_desc(b, row):
            return pltpu.make_async_copy(rows.at[pl.ds(b * S, S)],
                                         out_hbm.at[wid * RPW + row],
                                         sem_out.at[b])

        def stage_a(t, b):
            @pl.when(jnp.logical_and(t < RPW, t >= NB))
            def _():
                out_desc(b, t - NB).wait()

            @pl.when(t < RPW)
            def _():
                for d in tok_descs(b, t):
                    d.start()
                carry = jnp.int32(0)
                for part, nch in ((0, NCH0), (1, NCH1)):
                    r = 2 * t + part
                    for c in range(nch):
                        tok = idx_all[r, pl.ds(c * L, L)]
                        nz = jnp.where(tok != 0, 1, 0).astype(jnp.int32)
                        cs = _cumsum16(nz) + carry
                        posidx[2 * b + part, pl.ds(c * L, L)] = jnp.where(
                            tok == 0, 0, cs)
                        carry = cs[L - 1]

        def stage_b(j, b):
            @pl.when(jnp.logical_and(j >= 0, j < RPW))
            def _():
                for d in tok_descs(b, j):
                    d.wait()
                for d in pos_descs(b, j):
                    d.start(add=True)

        def stage_c(j, b):
            @pl.when(jnp.logical_and(j >= 0, j < RPW))
            def _():
                for d in pos_descs(b, j):
                    d.wait()
                out_desc(b, j).start()

        def step(g, dummy):
            for b in range(NB):
                t = g * NB + b
                stage_a(t, b)
                stage_b(t - 1, (b - 1) % NB)
                stage_c(t - 2, (b - 2) % NB)
            return dummy

        lax.fori_loop(0, (RPW + 2 + NB - 1) // NB, step, jnp.int32(0))
        for i in range(NB):
            j = RPW - NB + i
            out_desc(j % NB, j).wait()

    return sc_embed


def kernel(x, tok_table, pos_table):
    B, S = x.shape
    V, D = tok_table.shape
    P = pos_table.shape[0]
    xp = jnp.pad(x, ((0, 0), (0, 256 - S))).reshape(B * 2, 128)
    out = _make_sc_call(B, S, V, D, P)(xp, tok_table, pos_table)
    return out, (x == 0)
